# baseline (device time: 211984 ns/iter reference)
import jax
import jax.numpy as jnp
from jax import lax
from jax.experimental import pallas as pl
from jax.experimental.pallas import tpu as pltpu

N_DEV = 8
M, N = 4096, 2048
CHUNK = M // N_DEV
HALF = N // 2
LANES = 8
LCOL = N // LANES
LANE_DIR = (0, 1, 0, 1, 0, 1, 0, 1)
LANE_COL0 = (0, 1024, 256, 1280, 512, 1536, 768, 1792)


def kernel(x, w_mat, scale_x, scale_w):
    def body(x_ref, w_ref, sx_ref, sw_ref, out_ref,
             stage, rs_buf, rs_send_sems, rs_recv_sems, rs_credits,
             ag_buf, ag_send_sems, ag_recv_sems, ag_credits):
        my_pos = lax.axis_index("i")

        def ring_map(p):
            return jnp.where(p < 4, p, 11 - p)

        rank_cw = ring_map(my_pos)
        right = ring_map((rank_cw + 1) % N_DEV)
        left = ring_map((rank_cw - 1) % N_DEV)
        rank_ccw = (N_DEV - rank_cw) % N_DEV
        ranks = (rank_cw, rank_ccw)
        nxt = (right, left)
        prv = (left, right)

        def lane_rank(l):
            return ranks[LANE_DIR[l]]

        def lane_nxt(l):
            return nxt[LANE_DIR[l]]

        def lane_prv(l):
            return prv[LANE_DIR[l]]

        def cols(l):
            return pl.ds(LANE_COL0[l], LCOL)

        barrier_sem = pltpu.get_barrier_semaphore()
        for nbr in (left, right):
            pl.semaphore_signal(
                barrier_sem, inc=1,
                device_id=(nbr,), device_id_type=pl.DeviceIdType.MESH,
            )
        pl.semaphore_wait(barrier_sem, 2)

        s = (sx_ref[0] * sw_ref[0]).astype(jnp.float32)
        w_bf = w_ref[...].astype(jnp.bfloat16)

        def rs_send(l, h):
            rdma = pltpu.make_async_remote_copy(
                src_ref=stage.at[l],
                dst_ref=rs_buf.at[l, h % 2],
                send_sem=rs_send_sems.at[l, h % 2],
                recv_sem=rs_recv_sems.at[l, h % 2],
                device_id=(lane_nxt(l),),
                device_id_type=pl.DeviceIdType.MESH,
            )
            rdma.start()
            return rdma

        def ag_send(l, h):
            src = stage.at[l] if h == 0 else ag_buf.at[l, (h - 1) % 3]
            rdma = pltpu.make_async_remote_copy(
                src_ref=src,
                dst_ref=ag_buf.at[l, h % 3],
                send_sem=ag_send_sems.at[l, h % 3],
                recv_sem=ag_recv_sems.at[l, h % 3],
                device_id=(lane_nxt(l),),
                device_id_type=pl.DeviceIdType.MESH,
            )
            rdma.start()
            return rdma

        for c in range(N_DEV):
            rows = pl.ds(c * CHUNK, CHUNK)
            out_ref[rows, :] = (
                jnp.dot(x_ref[rows, :].astype(jnp.bfloat16), w_bf,
                        preferred_element_type=jnp.float32)
                * s
            )

        inflight = [None] * LANES
        for l in range(LANES):
            rows = pl.ds(lane_rank(l) * CHUNK, CHUNK)
            stage[l] = out_ref[rows, cols(l)].astype(jnp.bfloat16)
            inflight[l] = rs_send(l, 0)

        for h in range(N_DEV - 1):
            slot = h % 2
            for l in range(LANES):
                inflight[l].wait()
                recv_chunk = (lane_rank(l) - h - 1) % N_DEV
                rows = pl.ds(recv_chunk * CHUNK, CHUNK)
                acc = out_ref[rows, cols(l)] + rs_buf[l, slot].astype(jnp.float32)
                stage[l] = acc.astype(jnp.bfloat16)
                if h < N_DEV - 2:
                    if h + 1 >= 2:
                        pl.semaphore_wait(rs_credits.at[l, (h + 1) % 2], 1)
                    inflight[l] = rs_send(l, h + 1)
                else:
                    inflight[l] = ag_send(l, 0)
                    out_ref[rows, cols(l)] = acc
                if h <= 4:
                    pl.semaphore_signal(
                        rs_credits.at[l, slot], inc=1,
                        device_id=(lane_prv(l),),
                        device_id_type=pl.DeviceIdType.MESH,
                    )

        for h in range(N_DEV - 1):
            slot = h % 3
            for l in range(LANES):
                inflight[l].wait()
                if h < N_DEV - 2:
                    if h + 1 >= 3:
                        pl.semaphore_wait(ag_credits.at[l, (h + 1) % 3], 1)
                    inflight[l] = ag_send(l, h + 1)
                if 1 <= h <= 4:
                    pl.semaphore_signal(
                        ag_credits.at[l, (h - 1) % 3], inc=1,
                        device_id=(lane_prv(l),),
                        device_id_type=pl.DeviceIdType.MESH,
                    )
                recv_chunk = (lane_rank(l) - h) % N_DEV
                rows = pl.ds(recv_chunk * CHUNK, CHUNK)
                out_ref[rows, cols(l)] = ag_buf[l, slot].astype(jnp.float32)

    return pl.pallas_call(
        body,
        out_shape=jax.ShapeDtypeStruct((M, N), jnp.float32),
        in_specs=[
            pl.BlockSpec(memory_space=pltpu.VMEM),
            pl.BlockSpec(memory_space=pltpu.VMEM),
            pl.BlockSpec(memory_space=pltpu.VMEM),
            pl.BlockSpec(memory_space=pltpu.VMEM),
        ],
        out_specs=pl.BlockSpec(memory_space=pltpu.VMEM),
        scratch_shapes=[
            pltpu.VMEM((LANES, CHUNK, LCOL), jnp.bfloat16),
            pltpu.VMEM((LANES, 2, CHUNK, LCOL), jnp.bfloat16),
            pltpu.SemaphoreType.DMA((LANES, 2)),
            pltpu.SemaphoreType.DMA((LANES, 2)),
            pltpu.SemaphoreType.REGULAR((LANES, 2)),
            pltpu.VMEM((LANES, 3, CHUNK, LCOL), jnp.bfloat16),
            pltpu.SemaphoreType.DMA((LANES, 3)),
            pltpu.SemaphoreType.DMA((LANES, 3)),
            pltpu.SemaphoreType.REGULAR((LANES, 3)),
        ],
        compiler_params=pltpu.CompilerParams(
            collective_id=0,
            vmem_limit_bytes=60 * 1024 * 1024,
        ),
    )(x, w_mat, scale_x, scale_w)


# device time: 205927 ns/iter; 1.0294x vs baseline; 1.0294x over previous
import jax
import jax.numpy as jnp
from jax import lax
from jax.experimental import pallas as pl
from jax.experimental.pallas import tpu as pltpu

N_DEV = 8
M, N = 4096, 2048
CHUNK = M // N_DEV
HALF = N // 2
LANES = 8
LCOL = N // LANES
LANE_DIR = (0, 1, 0, 1, 0, 1, 0, 1)
LANE_COL0 = (0, 1024, 256, 1280, 512, 1536, 768, 1792)


def kernel(x, w_mat, scale_x, scale_w):
    def body(x_ref, w_ref, sx_ref, sw_ref, out_ref,
             stage, rs_buf, rs_send_sems, rs_recv_sems, rs_credits,
             ag_buf, ag_send_sems, ag_recv_sems, ag_credits):
        my_pos = lax.axis_index("i")

        def ring_map(p):
            return jnp.where(p < 4, p, 11 - p)

        rank_cw = ring_map(my_pos)
        right = ring_map((rank_cw + 1) % N_DEV)
        left = ring_map((rank_cw - 1) % N_DEV)
        rank_ccw = (N_DEV - rank_cw) % N_DEV
        ranks = (rank_cw, rank_ccw)
        nxt = (right, left)
        prv = (left, right)

        def lane_rank(l):
            return ranks[LANE_DIR[l]]

        def lane_nxt(l):
            return nxt[LANE_DIR[l]]

        def lane_prv(l):
            return prv[LANE_DIR[l]]

        def cols(l):
            return pl.ds(LANE_COL0[l], LCOL)

        barrier_sem = pltpu.get_barrier_semaphore()
        for nbr in (left, right):
            pl.semaphore_signal(
                barrier_sem, inc=1,
                device_id=(nbr,), device_id_type=pl.DeviceIdType.MESH,
            )
        pl.semaphore_wait(barrier_sem, 2)

        s = (sx_ref[0] * sw_ref[0]).astype(jnp.float32)
        w_bf = w_ref[...].astype(jnp.bfloat16)

        def rs_send(l, h):
            rdma = pltpu.make_async_remote_copy(
                src_ref=stage.at[l],
                dst_ref=rs_buf.at[l, h % 2],
                send_sem=rs_send_sems.at[l, h % 2],
                recv_sem=rs_recv_sems.at[l, h % 2],
                device_id=(lane_nxt(l),),
                device_id_type=pl.DeviceIdType.MESH,
            )
            rdma.start()
            return rdma

        def ag_send(l, h):
            src = stage.at[l] if h == 0 else ag_buf.at[l, (h - 1) % 3]
            rdma = pltpu.make_async_remote_copy(
                src_ref=src,
                dst_ref=ag_buf.at[l, h % 3],
                send_sem=ag_send_sems.at[l, h % 3],
                recv_sem=ag_recv_sems.at[l, h % 3],
                device_id=(lane_nxt(l),),
                device_id_type=pl.DeviceIdType.MESH,
            )
            rdma.start()
            return rdma

        is_own = [
            jnp.logical_or(jnp.int32(c) == rank_cw, jnp.int32(c) == rank_ccw)
            for c in range(N_DEV)
        ]

        def chunk_gemm(c):
            rows = pl.ds(c * CHUNK, CHUNK)
            out_ref[rows, :] = (
                jnp.dot(x_ref[rows, :].astype(jnp.bfloat16), w_bf,
                        preferred_element_type=jnp.float32)
                * s
            )

        for c in range(N_DEV):
            pl.when(is_own[c])(lambda c=c: chunk_gemm(c))

        inflight = [None] * LANES
        for l in range(LANES):
            rows = pl.ds(lane_rank(l) * CHUNK, CHUNK)
            stage[l] = out_ref[rows, cols(l)].astype(jnp.bfloat16)
            inflight[l] = rs_send(l, 0)

        for c in range(N_DEV):
            pl.when(jnp.logical_not(is_own[c]))(lambda c=c: chunk_gemm(c))

        for h in range(N_DEV - 1):
            slot = h % 2
            for l in range(LANES):
                inflight[l].wait()
                recv_chunk = (lane_rank(l) - h - 1) % N_DEV
                rows = pl.ds(recv_chunk * CHUNK, CHUNK)
                acc = out_ref[rows, cols(l)] + rs_buf[l, slot].astype(jnp.float32)
                stage[l] = acc.astype(jnp.bfloat16)
                if h < N_DEV - 2:
                    if h + 1 >= 2:
                        pl.semaphore_wait(rs_credits.at[l, (h + 1) % 2], 1)
                    inflight[l] = rs_send(l, h + 1)
                else:
                    inflight[l] = ag_send(l, 0)
                    out_ref[rows, cols(l)] = acc
                if h <= 4:
                    pl.semaphore_signal(
                        rs_credits.at[l, slot], inc=1,
                        device_id=(lane_prv(l),),
                        device_id_type=pl.DeviceIdType.MESH,
                    )

        for h in range(N_DEV - 1):
            slot = h % 3
            for l in range(LANES):
                inflight[l].wait()
                if h < N_DEV - 2:
                    if h + 1 >= 3:
                        pl.semaphore_wait(ag_credits.at[l, (h + 1) % 3], 1)
                    inflight[l] = ag_send(l, h + 1)
                if 1 <= h <= 4:
                    pl.semaphore_signal(
                        ag_credits.at[l, (h - 1) % 3], inc=1,
                        device_id=(lane_prv(l),),
                        device_id_type=pl.DeviceIdType.MESH,
                    )
                recv_chunk = (lane_rank(l) - h) % N_DEV
                rows = pl.ds(recv_chunk * CHUNK, CHUNK)
                out_ref[rows, cols(l)] = ag_buf[l, slot].astype(jnp.float32)

    return pl.pallas_call(
        body,
        out_shape=jax.ShapeDtypeStruct((M, N), jnp.float32),
        in_specs=[
            pl.BlockSpec(memory_space=pltpu.VMEM),
            pl.BlockSpec(memory_space=pltpu.VMEM),
            pl.BlockSpec(memory_space=pltpu.VMEM),
            pl.BlockSpec(memory_space=pltpu.VMEM),
        ],
        out_specs=pl.BlockSpec(memory_space=pltpu.VMEM),
        scratch_shapes=[
            pltpu.VMEM((LANES, CHUNK, LCOL), jnp.bfloat16),
            pltpu.VMEM((LANES, 2, CHUNK, LCOL), jnp.bfloat16),
            pltpu.SemaphoreType.DMA((LANES, 2)),
            pltpu.SemaphoreType.DMA((LANES, 2)),
            pltpu.SemaphoreType.REGULAR((LANES, 2)),
            pltpu.VMEM((LANES, 3, CHUNK, LCOL), jnp.bfloat16),
            pltpu.SemaphoreType.DMA((LANES, 3)),
            pltpu.SemaphoreType.DMA((LANES, 3)),
            pltpu.SemaphoreType.REGULAR((LANES, 3)),
        ],
        compiler_params=pltpu.CompilerParams(
            collective_id=0,
            vmem_limit_bytes=60 * 1024 * 1024,
        ),
    )(x, w_mat, scale_x, scale_w)
